# baseline (speedup 1.0000x reference)
import functools

import jax
import jax.numpy as jnp
from jax import lax
from jax.experimental import pallas as pl
from jax.experimental.pallas import tpu as pltpu
from jax.experimental.pallas import tpu_sc as plsc

IN_K = 32768
N_CH = 256
OUT_K = 8192
POS_PAD = 16

NC = 2
NS = 16
NW = NC * NS
BPW = OUT_K // NW


def _norm_body(w_ref, out_ref):
    w = w_ref[...]
    x2 = w * w
    t = x2[:, :128] + x2[:, 128:]
    tt = t.T.reshape(16, 8, t.shape[0])
    acc = tt[0]
    for j in range(1, 16):
        acc = acc + tt[j]
    p1 = acc[:4] + acc[4:]
    p2 = p1[:2] + p1[2:]
    p3 = p2[0] + p2[1]
    out_ref[...] = jnp.sqrt(p3)


def _row_norms(weights):
    R = 4096
    return pl.pallas_call(
        _norm_body,
        grid=(IN_K // R,),
        in_specs=[pl.BlockSpec((R, N_CH), lambda i: (i, 0))],
        out_specs=pl.BlockSpec((R,), lambda i: (i,)),
        out_shape=jax.ShapeDtypeStruct((IN_K,), jnp.float32),
    )(weights)


T_SORT = 16
CH = IN_K // T_SORT
RADIX = 2048
NPASS = 3
DBITS = 11
NVR = CH // 16

_mesh1 = plsc.VectorSubcoreMesh(core_axis_name="c", subcore_axis_name="s",
                                num_cores=1)


@functools.partial(
    pl.kernel,
    mesh=_mesh1,
    compiler_params=pltpu.CompilerParams(needs_layout_passes=False),
    out_type=[
        jax.ShapeDtypeStruct((OUT_K,), jnp.int32),
        jax.ShapeDtypeStruct((OUT_K * 3,), jnp.int32),
    ],
    scratch_types=[
        pltpu.VMEM((CH,), jnp.float32),
        pltpu.VMEM((CH,), jnp.uint32),
        pltpu.VMEM((CH,), jnp.int32),
        pltpu.VMEM((CH,), jnp.int32),
        pltpu.VMEM((RADIX,), jnp.int32),
        pltpu.VMEM((RADIX,), jnp.int32),
        pltpu.VMEM((T_SORT, RADIX), jnp.int32),
        pltpu.VMEM((CH * 3,), jnp.int32),
        pltpu.VMEM_SHARED((IN_K,), jnp.uint32),
        pltpu.VMEM_SHARED((IN_K,), jnp.int32),
        pltpu.VMEM_SHARED((IN_K,), jnp.uint32),
        pltpu.VMEM_SHARED((IN_K,), jnp.int32),
        pltpu.VMEM_SHARED((T_SORT, RADIX), jnp.int32),
        pltpu.SemaphoreType.DMA,
    ],
)
def _sc_sort(norms_hbm, oidx_hbm, oidx3_hbm,
             kvf, kv, iv, pv, offs, counter, histl, idx3v,
             KA, IA, KB, IB, HIST, sem):
    tid = lax.axis_index("s")
    base = tid * CH
    iota = lax.iota(jnp.int32, 16)

    def zero_counter():
        z = jnp.zeros((16,), jnp.int32)
        for c in range(RADIX // 16):
            counter[pl.ds(c * 16, 16)] = z

    def digit_of(k, p):
        sh = (DBITS * p).astype(jnp.uint32)
        return ((k >> sh) & jnp.uint32(RADIX - 1)).astype(jnp.int32)

    pltpu.sync_copy(norms_hbm.at[pl.ds(base, CH)], kvf)

    def pass_body(p, _):
        didx = p % 2
        sidx = 1 - didx

        zero_counter()

        @pl.when(p == 0)
        def _():
            def body_a(v, _):
                f = kvf[pl.ds(v * 16, 16)]
                k = ~plsc.bitcast(f, jnp.uint32)
                kv[pl.ds(v * 16, 16)] = k
                iv[pl.ds(v * 16, 16)] = base + v * 16 + iota
                d = digit_of(k, p)
                occ, last = plsc.scan_count(d)
                plsc.addupdate_scatter(counter, [d], occ, mask=last)
                return 0
            lax.fori_loop(0, NVR, body_a, 0)

        @pl.when(p > 0)
        def _():
            @pl.when(sidx == 0)
            def _():
                pltpu.sync_copy(KA.at[pl.ds(base, CH)], kv)
                pltpu.sync_copy(IA.at[pl.ds(base, CH)], iv)

            @pl.when(sidx == 1)
            def _():
                pltpu.sync_copy(KB.at[pl.ds(base, CH)], kv)
                pltpu.sync_copy(IB.at[pl.ds(base, CH)], iv)

            def body_a(v, _):
                k = kv[pl.ds(v * 16, 16)]
                d = digit_of(k, p)
                occ, last = plsc.scan_count(d)
                plsc.addupdate_scatter(counter, [d], occ, mask=last)
                return 0
            lax.fori_loop(0, NVR, body_a, 0)

        pltpu.sync_copy(counter, HIST.at[tid])
        plsc.subcore_barrier()

        pltpu.sync_copy(HIST, histl)

        def body_b(dc, carry):
            tot = jnp.zeros((16,), jnp.int32)
            par = jnp.zeros((16,), jnp.int32)
            for t in range(T_SORT):
                h = histl[t, pl.ds(dc * 16, 16)]
                tot = tot + h
                m = (jnp.int32(t) < tid).astype(jnp.int32)
                par = par + h * m
            cs = plsc.cumsum(tot)
            excl = cs - tot + carry
            offs[pl.ds(dc * 16, 16)] = excl + par
            return carry + jnp.sum(tot)
        lax.fori_loop(0, RADIX // 16, body_b, jnp.int32(0))

        zero_counter()

        def body_c(v, _):
            k = kv[pl.ds(v * 16, 16)]
            d = digit_of(k, p)
            occ, last = plsc.scan_count(d)
            cnt = plsc.load_gather(counter, [d])
            off = plsc.load_gather(offs, [d])
            pv[pl.ds(v * 16, 16)] = off + cnt + occ - 1
            plsc.addupdate_scatter(counter, [d], occ, mask=last)
            return 0
        lax.fori_loop(0, NVR, body_c, 0)

        @pl.when(didx == 0)
        def _():
            pltpu.sync_copy(kv, KA.at[pv])
            pltpu.sync_copy(iv, IA.at[pv])

        @pl.when(didx == 1)
        def _():
            pltpu.sync_copy(kv, KB.at[pv])
            pltpu.sync_copy(iv, IB.at[pv])
        plsc.subcore_barrier()
        return 0

    lax.fori_loop(0, NPASS, pass_body, 0)

    @pl.when(tid < OUT_K // CH)
    def _():
        pltpu.sync_copy(IA.at[pl.ds(base, CH)], iv)
        pltpu.sync_copy(iv, oidx_hbm.at[pl.ds(base, CH)])

        def body_f(v, _):
            i16 = iv[pl.ds(v * 16, 16)]
            for c in range(3):
                plsc.store_scatter(idx3v, [v * 48 + iota * 3 + c], i16 * 3 + c)
            return 0
        lax.fori_loop(0, NVR, body_f, 0)
        pltpu.sync_copy(idx3v, oidx3_hbm.at[pl.ds(base * 3, CH * 3)])


_mesh = plsc.VectorSubcoreMesh(core_axis_name="c", subcore_axis_name="s")


@functools.partial(
    pl.kernel,
    mesh=_mesh,
    out_type=[
        jax.ShapeDtypeStruct((OUT_K * 3,), jnp.float32),
        jax.ShapeDtypeStruct((OUT_K, N_CH), jnp.float32),
    ],
    scratch_types=[
        pltpu.VMEM((BPW,), jnp.int32),
        pltpu.VMEM((BPW * 3,), jnp.int32),
        pltpu.VMEM((BPW * 3,), jnp.float32),
        pltpu.VMEM((BPW, N_CH), jnp.float32),
        pltpu.SemaphoreType.DMA,
    ],
)
def _sc_gather(idx_hbm, idx3_hbm, pos_hbm, w_hbm, opos_hbm, ow_hbm,
               idx_v, idx3_v, pvals_v, wrows_v, sem):
    wid = lax.axis_index("s") * NC + lax.axis_index("c")
    base = wid * BPW
    pltpu.sync_copy(idx_hbm.at[pl.ds(base, BPW)], idx_v)
    pltpu.sync_copy(idx3_hbm.at[pl.ds(base * 3, BPW * 3)], idx3_v)
    pltpu.async_copy(pos_hbm.at[idx3_v], pvals_v, sem).wait()
    pltpu.sync_copy(pvals_v, opos_hbm.at[pl.ds(base * 3, BPW * 3)])
    pltpu.async_copy(w_hbm.at[idx_v], wrows_v, sem).wait()
    pltpu.sync_copy(wrows_v, ow_hbm.at[pl.ds(base, BPW)])


def kernel(positions, weights):
    norms = _row_norms(weights)
    idx, idx3 = _sc_sort(norms)
    pos_flat = positions.reshape(-1)
    opos_flat, ow = _sc_gather(idx, idx3, pos_flat, weights)
    return opos_flat.reshape(OUT_K, 3), ow


# speedup vs baseline: 1.0608x; 1.0608x over previous
import functools

import jax
import jax.numpy as jnp
from jax import lax
from jax.experimental import pallas as pl
from jax.experimental.pallas import tpu as pltpu
from jax.experimental.pallas import tpu_sc as plsc

IN_K = 32768
N_CH = 256
OUT_K = 8192
POS_PAD = 16

NC = 2
NS = 16
NW = NC * NS
BPW = OUT_K // NW


def _norm_body(w_ref, out_ref):
    w = w_ref[...]
    x2 = w * w
    t = x2[:, :128] + x2[:, 128:]
    tt = t.T.reshape(16, 8, t.shape[0])
    acc = tt[0]
    for j in range(1, 16):
        acc = acc + tt[j]
    p1 = acc[:4] + acc[4:]
    p2 = p1[:2] + p1[2:]
    p3 = p2[0] + p2[1]
    out_ref[...] = jnp.sqrt(p3)


def _row_norms(weights):
    R = 8192
    return pl.pallas_call(
        _norm_body,
        grid=(IN_K // R,),
        in_specs=[pl.BlockSpec((R, N_CH), lambda i: (i, 0))],
        out_specs=pl.BlockSpec((R,), lambda i: (i,)),
        out_shape=jax.ShapeDtypeStruct((IN_K,), jnp.float32),
    )(weights)


T_SORT = 16
CH = IN_K // T_SORT
RADIX = 256
NPASS = 4
DBITS = 8
NVR = CH // 16

_mesh1 = plsc.VectorSubcoreMesh(core_axis_name="c", subcore_axis_name="s",
                                num_cores=1)


@functools.partial(
    pl.kernel,
    mesh=_mesh1,
    compiler_params=pltpu.CompilerParams(needs_layout_passes=False),
    out_type=[
        jax.ShapeDtypeStruct((OUT_K,), jnp.int32),
        jax.ShapeDtypeStruct((OUT_K * 3,), jnp.int32),
    ],
    scratch_types=[
        pltpu.VMEM((CH,), jnp.float32),
        pltpu.VMEM((CH,), jnp.uint32),
        pltpu.VMEM((CH,), jnp.int32),
        pltpu.VMEM((CH,), jnp.int32),
        pltpu.VMEM((RADIX,), jnp.int32),
        pltpu.VMEM((RADIX,), jnp.int32),
        pltpu.VMEM((T_SORT, RADIX), jnp.int32),
        pltpu.VMEM((CH * 3,), jnp.int32),
        pltpu.VMEM_SHARED((IN_K,), jnp.uint32),
        pltpu.VMEM_SHARED((IN_K,), jnp.int32),
        pltpu.VMEM_SHARED((IN_K,), jnp.uint32),
        pltpu.VMEM_SHARED((IN_K,), jnp.int32),
        pltpu.VMEM_SHARED((T_SORT, RADIX), jnp.int32),
        pltpu.SemaphoreType.DMA,
    ],
)
def _sc_sort(norms_hbm, oidx_hbm, oidx3_hbm,
             kvf, kv, iv, pv, offs, counter, histl, idx3v,
             KA, IA, KB, IB, HIST, sem):
    tid = lax.axis_index("s")
    base = tid * CH
    iota = lax.iota(jnp.int32, 16)

    def zero_counter():
        z = jnp.zeros((16,), jnp.int32)
        for c in range(RADIX // 16):
            counter[pl.ds(c * 16, 16)] = z

    def digit_of(k, p):
        sh = (DBITS * p).astype(jnp.uint32)
        return ((k >> sh) & jnp.uint32(RADIX - 1)).astype(jnp.int32)

    pltpu.sync_copy(norms_hbm.at[pl.ds(base, CH)], kvf)

    def pass_body(p, _):
        didx = p % 2
        sidx = 1 - didx

        zero_counter()

        @pl.when(p == 0)
        def _():
            def body_a(v, _):
                f = kvf[pl.ds(v * 16, 16)]
                k = ~plsc.bitcast(f, jnp.uint32)
                kv[pl.ds(v * 16, 16)] = k
                iv[pl.ds(v * 16, 16)] = base + v * 16 + iota
                d = digit_of(k, p)
                occ, last = plsc.scan_count(d)
                plsc.addupdate_scatter(counter, [d], occ, mask=last)
                return 0
            lax.fori_loop(0, NVR, body_a, 0)

        @pl.when(p > 0)
        def _():
            @pl.when(sidx == 0)
            def _():
                pltpu.sync_copy(KA.at[pl.ds(base, CH)], kv)
                pltpu.sync_copy(IA.at[pl.ds(base, CH)], iv)

            @pl.when(sidx == 1)
            def _():
                pltpu.sync_copy(KB.at[pl.ds(base, CH)], kv)
                pltpu.sync_copy(IB.at[pl.ds(base, CH)], iv)

            def body_a(v, _):
                k = kv[pl.ds(v * 16, 16)]
                d = digit_of(k, p)
                occ, last = plsc.scan_count(d)
                plsc.addupdate_scatter(counter, [d], occ, mask=last)
                return 0
            lax.fori_loop(0, NVR, body_a, 0)

        pltpu.sync_copy(counter, HIST.at[tid])
        plsc.subcore_barrier()

        pltpu.sync_copy(HIST, histl)

        def body_b(dc, carry):
            tot = jnp.zeros((16,), jnp.int32)
            par = jnp.zeros((16,), jnp.int32)
            for t in range(T_SORT):
                h = histl[t, pl.ds(dc * 16, 16)]
                tot = tot + h
                m = (jnp.int32(t) < tid).astype(jnp.int32)
                par = par + h * m
            cs = plsc.cumsum(tot)
            excl = cs - tot + carry
            offs[pl.ds(dc * 16, 16)] = excl + par
            return carry + jnp.sum(tot)
        lax.fori_loop(0, RADIX // 16, body_b, jnp.int32(0))

        zero_counter()

        def body_c(v, _):
            k = kv[pl.ds(v * 16, 16)]
            d = digit_of(k, p)
            occ, last = plsc.scan_count(d)
            cnt = plsc.load_gather(counter, [d])
            off = plsc.load_gather(offs, [d])
            pv[pl.ds(v * 16, 16)] = off + cnt + occ - 1
            plsc.addupdate_scatter(counter, [d], occ, mask=last)
            return 0
        lax.fori_loop(0, NVR, body_c, 0)

        @pl.when(didx == 0)
        def _():
            pltpu.sync_copy(kv, KA.at[pv])
            pltpu.sync_copy(iv, IA.at[pv])

        @pl.when(didx == 1)
        def _():
            pltpu.sync_copy(kv, KB.at[pv])
            pltpu.sync_copy(iv, IB.at[pv])
        plsc.subcore_barrier()
        return 0

    lax.fori_loop(0, NPASS, pass_body, 0)

    @pl.when(tid < OUT_K // CH)
    def _():
        pltpu.sync_copy(IB.at[pl.ds(base, CH)], iv)
        pltpu.sync_copy(iv, oidx_hbm.at[pl.ds(base, CH)])

        def body_f(v, _):
            i16 = iv[pl.ds(v * 16, 16)]
            for c in range(3):
                plsc.store_scatter(idx3v, [v * 48 + iota * 3 + c], i16 * 3 + c)
            return 0
        lax.fori_loop(0, NVR, body_f, 0)
        pltpu.sync_copy(idx3v, oidx3_hbm.at[pl.ds(base * 3, CH * 3)])


_mesh = plsc.VectorSubcoreMesh(core_axis_name="c", subcore_axis_name="s")


@functools.partial(
    pl.kernel,
    mesh=_mesh,
    out_type=[
        jax.ShapeDtypeStruct((OUT_K * 3,), jnp.float32),
        jax.ShapeDtypeStruct((OUT_K, N_CH), jnp.float32),
    ],
    scratch_types=[
        pltpu.VMEM((BPW,), jnp.int32),
        pltpu.VMEM((BPW * 3,), jnp.int32),
        pltpu.VMEM((BPW * 3,), jnp.float32),
        pltpu.VMEM((BPW, N_CH), jnp.float32),
        pltpu.SemaphoreType.DMA,
    ],
)
def _sc_gather(idx_hbm, idx3_hbm, pos_hbm, w_hbm, opos_hbm, ow_hbm,
               idx_v, idx3_v, pvals_v, wrows_v, sem):
    wid = lax.axis_index("s") * NC + lax.axis_index("c")
    base = wid * BPW
    pltpu.sync_copy(idx_hbm.at[pl.ds(base, BPW)], idx_v)
    pltpu.sync_copy(idx3_hbm.at[pl.ds(base * 3, BPW * 3)], idx3_v)
    pltpu.async_copy(pos_hbm.at[idx3_v], pvals_v, sem).wait()
    pltpu.sync_copy(pvals_v, opos_hbm.at[pl.ds(base * 3, BPW * 3)])
    pltpu.async_copy(w_hbm.at[idx_v], wrows_v, sem).wait()
    pltpu.sync_copy(wrows_v, ow_hbm.at[pl.ds(base, BPW)])


def kernel(positions, weights):
    norms = _row_norms(weights)
    idx, idx3 = _sc_sort(norms)
    pos_flat = positions.reshape(-1)
    opos_flat, ow = _sc_gather(idx, idx3, pos_flat, weights)
    return opos_flat.reshape(OUT_K, 3), ow


# speedup vs baseline: 1.0910x; 1.0284x over previous
import functools

import jax
import jax.numpy as jnp
from jax import lax
from jax.experimental import pallas as pl
from jax.experimental.pallas import tpu as pltpu
from jax.experimental.pallas import tpu_sc as plsc

IN_K = 32768
N_CH = 256
OUT_K = 8192
POS_PAD = 16

NC = 2
NS = 16
NW = NC * NS
BPW = OUT_K // NW


def _norm_body(w_ref, out_ref):
    w = w_ref[...]
    x2 = w * w
    t = x2[:, :128] + x2[:, 128:]
    tt = t.T.reshape(16, 8, t.shape[0])
    acc = tt[0]
    for j in range(1, 16):
        acc = acc + tt[j]
    p1 = acc[:4] + acc[4:]
    p2 = p1[:2] + p1[2:]
    p3 = p2[0] + p2[1]
    out_ref[...] = jnp.sqrt(p3)


def _row_norms(weights):
    R = 8192
    return pl.pallas_call(
        _norm_body,
        grid=(IN_K // R,),
        in_specs=[pl.BlockSpec((R, N_CH), lambda i: (i, 0))],
        out_specs=pl.BlockSpec((R,), lambda i: (i,)),
        out_shape=jax.ShapeDtypeStruct((IN_K,), jnp.float32),
    )(weights)


T_SORT = 16
CH = IN_K // T_SORT
RADIX = 256
NPASS = 4
DBITS = 8
NVR = CH // 16

_mesh1 = plsc.VectorSubcoreMesh(core_axis_name="c", subcore_axis_name="s",
                                num_cores=1)


@functools.partial(
    pl.kernel,
    mesh=_mesh1,
    compiler_params=pltpu.CompilerParams(needs_layout_passes=False),
    out_type=[
        jax.ShapeDtypeStruct((OUT_K,), jnp.int32),
        jax.ShapeDtypeStruct((OUT_K * 3,), jnp.int32),
    ],
    scratch_types=[
        pltpu.VMEM((CH,), jnp.float32),
        pltpu.VMEM((CH,), jnp.uint32),
        pltpu.VMEM((CH,), jnp.int32),
        pltpu.VMEM((CH,), jnp.int32),
        pltpu.VMEM((CH,), jnp.int32),
        pltpu.VMEM((CH,), jnp.int32),
        pltpu.VMEM((RADIX,), jnp.int32),
        pltpu.VMEM((RADIX,), jnp.int32),
        pltpu.VMEM((T_SORT, RADIX), jnp.int32),
        pltpu.VMEM((CH * 3,), jnp.int32),
        pltpu.VMEM_SHARED((IN_K,), jnp.uint32),
        pltpu.VMEM_SHARED((IN_K,), jnp.int32),
        pltpu.VMEM_SHARED((IN_K,), jnp.uint32),
        pltpu.VMEM_SHARED((IN_K,), jnp.int32),
        pltpu.VMEM_SHARED((T_SORT, RADIX), jnp.int32),
        pltpu.SemaphoreType.DMA,
    ],
)
def _sc_sort(norms_hbm, oidx_hbm, oidx3_hbm,
             kvf, kv, iv, pv, ov, lv, offs, counter, histl, idx3v,
             KA, IA, KB, IB, HIST, sem):
    tid = lax.axis_index("s")
    base = tid * CH
    iota = lax.iota(jnp.int32, 16)

    def zero_counter():
        z = jnp.zeros((16,), jnp.int32)
        for c in range(RADIX // 16):
            counter[pl.ds(c * 16, 16)] = z

    def digit_of(k, p):
        sh = (DBITS * p).astype(jnp.uint32)
        return ((k >> sh) & jnp.uint32(RADIX - 1)).astype(jnp.int32)

    pltpu.sync_copy(norms_hbm.at[pl.ds(base, CH)], kvf)

    def pass_body(p, _):
        didx = p % 2
        sidx = 1 - didx

        zero_counter()

        @pl.when(p == 0)
        def _():
            def body_a(v, _):
                f = kvf[pl.ds(v * 16, 16)]
                k = ~plsc.bitcast(f, jnp.uint32)
                kv[pl.ds(v * 16, 16)] = k
                iv[pl.ds(v * 16, 16)] = base + v * 16 + iota
                d = digit_of(k, p)
                occ, last = plsc.scan_count(d)
                ov[pl.ds(v * 16, 16)] = occ
                lv[pl.ds(v * 16, 16)] = last.astype(jnp.int32)
                plsc.addupdate_scatter(counter, [d], occ, mask=last)
                return 0
            lax.fori_loop(0, NVR, body_a, 0)

        @pl.when(p > 0)
        def _():
            @pl.when(sidx == 0)
            def _():
                pltpu.sync_copy(KA.at[pl.ds(base, CH)], kv)
                pltpu.sync_copy(IA.at[pl.ds(base, CH)], iv)

            @pl.when(sidx == 1)
            def _():
                pltpu.sync_copy(KB.at[pl.ds(base, CH)], kv)
                pltpu.sync_copy(IB.at[pl.ds(base, CH)], iv)

            def body_a(v, _):
                k = kv[pl.ds(v * 16, 16)]
                d = digit_of(k, p)
                occ, last = plsc.scan_count(d)
                ov[pl.ds(v * 16, 16)] = occ
                lv[pl.ds(v * 16, 16)] = last.astype(jnp.int32)
                plsc.addupdate_scatter(counter, [d], occ, mask=last)
                return 0
            lax.fori_loop(0, NVR, body_a, 0)

        pltpu.sync_copy(counter, HIST.at[tid])
        plsc.subcore_barrier()

        pltpu.sync_copy(HIST, histl)

        def body_b(dc, carry):
            tot = jnp.zeros((16,), jnp.int32)
            par = jnp.zeros((16,), jnp.int32)
            for t in range(T_SORT):
                h = histl[t, pl.ds(dc * 16, 16)]
                tot = tot + h
                m = (jnp.int32(t) < tid).astype(jnp.int32)
                par = par + h * m
            cs = plsc.cumsum(tot)
            excl = cs - tot + carry
            offs[pl.ds(dc * 16, 16)] = excl + par
            return carry + jnp.sum(tot)
        lax.fori_loop(0, RADIX // 16, body_b, jnp.int32(0))

        zero_counter()

        def body_c(v, _):
            for u in (2 * v, 2 * v + 1):
                k = kv[pl.ds(u * 16, 16)]
                d = digit_of(k, p)
                occ = ov[pl.ds(u * 16, 16)]
                last = lv[pl.ds(u * 16, 16)] == 1
                cnt = plsc.load_gather(counter, [d])
                off = plsc.load_gather(offs, [d])
                pv[pl.ds(u * 16, 16)] = off + cnt + occ - 1
                plsc.addupdate_scatter(counter, [d], occ, mask=last)
            return 0
        lax.fori_loop(0, NVR // 2, body_c, 0)

        @pl.when(didx == 0)
        def _():
            pltpu.sync_copy(kv, KA.at[pv])
            pltpu.sync_copy(iv, IA.at[pv])

        @pl.when(didx == 1)
        def _():
            pltpu.sync_copy(kv, KB.at[pv])
            pltpu.sync_copy(iv, IB.at[pv])
        plsc.subcore_barrier()
        return 0

    lax.fori_loop(0, NPASS, pass_body, 0)

    @pl.when(tid < OUT_K // CH)
    def _():
        pltpu.sync_copy(IB.at[pl.ds(base, CH)], iv)
        pltpu.sync_copy(iv, oidx_hbm.at[pl.ds(base, CH)])

        def body_f(v, _):
            i16 = iv[pl.ds(v * 16, 16)]
            for c in range(3):
                plsc.store_scatter(idx3v, [v * 48 + iota * 3 + c], i16 * 3 + c)
            return 0
        lax.fori_loop(0, NVR, body_f, 0)
        pltpu.sync_copy(idx3v, oidx3_hbm.at[pl.ds(base * 3, CH * 3)])


_mesh = plsc.VectorSubcoreMesh(core_axis_name="c", subcore_axis_name="s")


@functools.partial(
    pl.kernel,
    mesh=_mesh,
    out_type=[
        jax.ShapeDtypeStruct((OUT_K * 3,), jnp.float32),
        jax.ShapeDtypeStruct((OUT_K, N_CH), jnp.float32),
    ],
    scratch_types=[
        pltpu.VMEM((BPW,), jnp.int32),
        pltpu.VMEM((BPW * 3,), jnp.int32),
        pltpu.VMEM((BPW * 3,), jnp.float32),
        pltpu.VMEM((BPW, N_CH), jnp.float32),
        pltpu.SemaphoreType.DMA,
    ],
)
def _sc_gather(idx_hbm, idx3_hbm, pos_hbm, w_hbm, opos_hbm, ow_hbm,
               idx_v, idx3_v, pvals_v, wrows_v, sem):
    wid = lax.axis_index("s") * NC + lax.axis_index("c")
    base = wid * BPW
    pltpu.sync_copy(idx_hbm.at[pl.ds(base, BPW)], idx_v)
    pltpu.sync_copy(idx3_hbm.at[pl.ds(base * 3, BPW * 3)], idx3_v)
    pltpu.async_copy(pos_hbm.at[idx3_v], pvals_v, sem).wait()
    pltpu.sync_copy(pvals_v, opos_hbm.at[pl.ds(base * 3, BPW * 3)])
    pltpu.async_copy(w_hbm.at[idx_v], wrows_v, sem).wait()
    pltpu.sync_copy(wrows_v, ow_hbm.at[pl.ds(base, BPW)])


def kernel(positions, weights):
    norms = _row_norms(weights)
    idx, idx3 = _sc_sort(norms)
    pos_flat = positions.reshape(-1)
    opos_flat, ow = _sc_gather(idx, idx3, pos_flat, weights)
    return opos_flat.reshape(OUT_K, 3), ow
